# baseline (speedup 1.0000x reference)
import functools
import math

import jax
import jax.numpy as jnp
from jax import lax
from jax.experimental import pallas as pl
from jax.experimental.pallas import tpu as pltpu
from jax.experimental.pallas import tpu_sc as plsc

S = 2048
D = 1024
NHEAD = 16
DH = 64
E = 8
TOPK = 2
DFF = 4096
CAP = 640
TRASH = E * CAP
BUFN = TRASH + 128

ENT = TOPK * S
NW = 32
EPW = ENT // NW
CH = 64
NCH = EPW // CH

BS = 256
NEG = -1e30


def _ln(x, g, b):
    m = jnp.mean(x, axis=-1, keepdims=True)
    v = jnp.mean((x - m) ** 2, axis=-1, keepdims=True)
    return (x - m) * jax.lax.rsqrt(v + 1e-5) * g + b


def _k1_body(x_ref, g_ref, b_ref, w_ref, o_ref):
    xn = _ln(x_ref[...], g_ref[...], b_ref[...])
    o_ref[...] = jnp.dot(xn, w_ref[...], preferred_element_type=jnp.float32)


def _ln_qkv(x, g, b, w):
    return pl.pallas_call(
        _k1_body,
        grid=(S // BS, (3 * D) // 512),
        in_specs=[
            pl.BlockSpec((BS, D), lambda i, j: (i, 0)),
            pl.BlockSpec((1, D), lambda i, j: (0, 0)),
            pl.BlockSpec((1, D), lambda i, j: (0, 0)),
            pl.BlockSpec((D, 512), lambda i, j: (0, j)),
        ],
        out_specs=pl.BlockSpec((BS, 512), lambda i, j: (i, j)),
        out_shape=jax.ShapeDtypeStruct((S, 3 * D), jnp.float32),
    )(x, g, b, w)


def _k2_body(q_ref, k_ref, v_ref, o_ref):
    qi = pl.program_id(1)
    q = q_ref[0] * (1.0 / math.sqrt(DH))
    row_ids = qi * BS + lax.broadcasted_iota(jnp.int32, (BS, BS), 0)
    col_base = lax.broadcasted_iota(jnp.int32, (BS, BS), 1)

    def step_fixed(kb, carry):
        m, l, acc = carry
        kk = k_ref[0, pl.ds(kb * BS, BS), :]
        vv = v_ref[0, pl.ds(kb * BS, BS), :]
        s = lax.dot_general(q, kk, (((1,), (1,)), ((), ())),
                            preferred_element_type=jnp.float32)
        mask = (kb * BS + col_base) <= row_ids
        s = jnp.where(mask, s, NEG)
        m_new = jnp.maximum(m, jnp.max(s, axis=1, keepdims=True))
        p = jnp.exp(s - m_new)
        corr = jnp.exp(m - m_new)
        l_new = l * corr + jnp.sum(p, axis=1, keepdims=True)
        acc_new = acc * corr + jnp.dot(p, vv,
                                       preferred_element_type=jnp.float32)
        return m_new, l_new, acc_new

    init = (jnp.full((BS, 1), NEG, jnp.float32),
            jnp.zeros((BS, 1), jnp.float32),
            jnp.zeros((BS, DH), jnp.float32))
    m, l, acc = lax.fori_loop(0, qi + 1, step_fixed, init)
    o_ref[0] = acc / l


def _attention(q, k, v):
    return pl.pallas_call(
        _k2_body,
        grid=(NHEAD, S // BS),
        in_specs=[
            pl.BlockSpec((1, BS, DH), lambda h, i: (h, i, 0)),
            pl.BlockSpec((1, S, DH), lambda h, i: (h, 0, 0)),
            pl.BlockSpec((1, S, DH), lambda h, i: (h, 0, 0)),
        ],
        out_specs=pl.BlockSpec((1, BS, DH), lambda h, i: (h, i, 0)),
        out_shape=jax.ShapeDtypeStruct((NHEAD, S, DH), jnp.float32),
    )(q, k, v)


def _k3_body(a_ref, w_ref, x_ref, o_ref):
    o_ref[...] = x_ref[...] + jnp.dot(a_ref[...], w_ref[...],
                                      preferred_element_type=jnp.float32)


def _proj_res(attn, w_o, x):
    return pl.pallas_call(
        _k3_body,
        grid=(S // BS, D // 256),
        in_specs=[
            pl.BlockSpec((BS, D), lambda i, j: (i, 0)),
            pl.BlockSpec((D, 256), lambda i, j: (0, j)),
            pl.BlockSpec((BS, 256), lambda i, j: (i, j)),
        ],
        out_specs=pl.BlockSpec((BS, 256), lambda i, j: (i, j)),
        out_shape=jax.ShapeDtypeStruct((S, D), jnp.float32),
    )(attn, w_o, x)


def _k4_body(x_ref, g_ref, b_ref, wr_ref, h_ref, lg_ref):
    hn = _ln(x_ref[...], g_ref[...], b_ref[...])
    h_ref[...] = hn
    lg_ref[...] = jnp.dot(hn, wr_ref[...], preferred_element_type=jnp.float32)


def _ln2_router(x1, g, b, wr_pad):
    return pl.pallas_call(
        _k4_body,
        grid=(S // BS,),
        in_specs=[
            pl.BlockSpec((BS, D), lambda i: (i, 0)),
            pl.BlockSpec((1, D), lambda i: (0, 0)),
            pl.BlockSpec((1, D), lambda i: (0, 0)),
            pl.BlockSpec((D, 128), lambda i: (0, 0)),
        ],
        out_specs=[
            pl.BlockSpec((BS, D), lambda i: (i, 0)),
            pl.BlockSpec((BS, 128), lambda i: (i, 0)),
        ],
        out_shape=[
            jax.ShapeDtypeStruct((S, D), jnp.float32),
            jax.ShapeDtypeStruct((S, 128), jnp.float32),
        ],
    )(x1, g, b, wr_pad)


def _k5_body(lg_ref, slot_ref, gk_ref):
    lane = lax.broadcasted_iota(jnp.int32, (BS, 128), 1)
    tril = (lax.broadcasted_iota(jnp.int32, (BS, BS), 0)
            > lax.broadcasted_iota(jnp.int32, (BS, BS), 1)).astype(jnp.float32)
    counts = jnp.zeros((1, 128), jnp.float32)
    for c in range(TOPK):
        for b in range(S // BS):
            raw = lg_ref[pl.ds(b * BS, BS), :]
            lg = jnp.where(lane < E, raw, NEG)
            m1 = jnp.max(lg, axis=1, keepdims=True)
            i1 = jnp.min(jnp.where(lg == m1, lane, 1000), axis=1,
                         keepdims=True)
            lg2 = jnp.where(lane == i1, NEG, lg)
            m2 = jnp.max(lg2, axis=1, keepdims=True)
            i2 = jnp.min(jnp.where(lg2 == m2, lane, 1000), axis=1,
                         keepdims=True)
            e2x = jnp.exp(m2 - m1)
            if c == 0:
                ei, gate = i1, 1.0 / (1.0 + e2x)
            else:
                ei, gate = i2, e2x / (1.0 + e2x)
            onehot = (lane == ei).astype(jnp.float32)
            run = jnp.dot(tril, onehot, preferred_element_type=jnp.float32)
            pos = jnp.sum(onehot * (counts + run), axis=1, keepdims=True)
            keep = pos < CAP
            pos_i = pos.astype(jnp.int32)
            slot = jnp.where(keep, ei * CAP + pos_i, TRASH)
            slot_ref[pl.ds(b * BS, BS), c:c + 1] = slot
            gk_ref[pl.ds(b * BS, BS), c:c + 1] = jnp.where(keep, gate, 0.0)
            counts = counts + jnp.sum(onehot, axis=0, keepdims=True)


def _routing(logits):
    return pl.pallas_call(
        _k5_body,
        in_specs=[pl.BlockSpec((S, 128), lambda: (0, 0))],
        out_specs=[
            pl.BlockSpec((S, TOPK), lambda: (0, 0)),
            pl.BlockSpec((S, TOPK), lambda: (0, 0)),
        ],
        out_shape=[
            jax.ShapeDtypeStruct((S, TOPK), jnp.int32),
            jax.ShapeDtypeStruct((S, TOPK), jnp.float32),
        ],
    )(logits)


def _sc_dispatch_body(h_hbm, slot_hbm, buf_hbm, idx_v, rows_v, sem):
    wid = lax.axis_index("s") * 2 + lax.axis_index("c")
    for ch in range(NCH):
        base = wid * EPW + ch * CH
        tok = base % S
        pltpu.sync_copy(slot_hbm.at[wid, ch], idx_v)
        pltpu.sync_copy(h_hbm.at[pl.ds(tok, CH)], rows_v)
        pltpu.async_copy(rows_v, buf_hbm.at[idx_v], sem).wait()


def _sc_dispatch(h, slot3d):
    kfn = pl.kernel(
        _sc_dispatch_body,
        out_type=jax.ShapeDtypeStruct((BUFN, D), jnp.float32),
        scratch_types=[
            pltpu.VMEM((CH,), jnp.int32),
            pltpu.VMEM((CH, D), jnp.float32),
            pltpu.SemaphoreType.DMA,
        ],
        mesh=plsc.VectorSubcoreMesh(core_axis_name="c", subcore_axis_name="s"),
    )
    return kfn(h, slot3d)


def _k7_body(buf_ref, w1_ref, w2_ref, o_ref, h1_ref):
    f = pl.program_id(1)
    h1 = jnp.dot(buf_ref[...], w1_ref[0],
                 preferred_element_type=jnp.float32)
    h1_ref[...] = jax.nn.gelu(h1, approximate=True)

    @pl.when(f == 0)
    def _():
        o_ref[...] = jnp.zeros_like(o_ref)

    o_ref[...] += jnp.dot(h1_ref[...], w2_ref[0],
                          preferred_element_type=jnp.float32)


def _expert_ffn(buf, w1, w2):
    fsp = DFF // 2
    return pl.pallas_call(
        _k7_body,
        grid=(E, 2),
        in_specs=[
            pl.BlockSpec((CAP, D), lambda e, f: (e, 0)),
            pl.BlockSpec((1, D, fsp), lambda e, f: (e, 0, f)),
            pl.BlockSpec((1, fsp, D), lambda e, f: (e, f, 0)),
        ],
        out_specs=pl.BlockSpec((CAP, D), lambda e, f: (e, 0)),
        out_shape=jax.ShapeDtypeStruct((BUFN, D), jnp.float32),
        scratch_shapes=[pltpu.VMEM((CAP, fsp), jnp.float32)],
    )(buf, w1, w2)


def _sc_combine_body(oute_hbm, slot_hbm, ycat_hbm, idx_v, rows_v, sem):
    wid = lax.axis_index("s") * 2 + lax.axis_index("c")
    for ch in range(NCH):
        base = wid * EPW + ch * CH
        pltpu.sync_copy(slot_hbm.at[wid, ch], idx_v)
        pltpu.async_copy(oute_hbm.at[idx_v], rows_v, sem).wait()
        pltpu.sync_copy(rows_v, ycat_hbm.at[pl.ds(base, CH)])


def _sc_combine(out_ext, slot3d):
    kfn = pl.kernel(
        _sc_combine_body,
        out_type=jax.ShapeDtypeStruct((ENT, D), jnp.float32),
        scratch_types=[
            pltpu.VMEM((CH,), jnp.int32),
            pltpu.VMEM((CH, D), jnp.float32),
            pltpu.SemaphoreType.DMA,
        ],
        mesh=plsc.VectorSubcoreMesh(core_axis_name="c", subcore_axis_name="s"),
    )
    return kfn(out_ext, slot3d)


def _k9_body(x_ref, y1_ref, y2_ref, g1_ref, g2_ref, o_ref):
    g1 = g1_ref[...][:, 0:1]
    g2 = g2_ref[...][:, 0:1]
    y1 = jnp.where(g1 > 0, y1_ref[...] * g1, 0.0)
    y2 = jnp.where(g2 > 0, y2_ref[...] * g2, 0.0)
    o_ref[...] = x_ref[...] + y1 + y2


def _combine(x1, y1, y2, g1, g2):
    return pl.pallas_call(
        _k9_body,
        grid=(S // BS,),
        in_specs=[
            pl.BlockSpec((BS, D), lambda i: (i, 0)),
            pl.BlockSpec((BS, D), lambda i: (i, 0)),
            pl.BlockSpec((BS, D), lambda i: (i, 0)),
            pl.BlockSpec((BS, 128), lambda i: (i, 0)),
            pl.BlockSpec((BS, 128), lambda i: (i, 0)),
        ],
        out_specs=pl.BlockSpec((BS, D), lambda i: (i, 0)),
        out_shape=jax.ShapeDtypeStruct((S, D), jnp.float32),
    )(x1, y1, y2, g1, g2)


def kernel(x, ln1_g, ln1_b, W_qkv, W_o, ln2_g, ln2_b, W_r, W1, W2):
    x2d = x.reshape(S, D)
    g1r = ln1_g.reshape(1, D)
    b1r = ln1_b.reshape(1, D)
    g2r = ln2_g.reshape(1, D)
    b2r = ln2_b.reshape(1, D)

    qkv = _ln_qkv(x2d, g1r, b1r, W_qkv)
    q = qkv[:, :D].reshape(S, NHEAD, DH).transpose(1, 0, 2)
    k = qkv[:, D:2 * D].reshape(S, NHEAD, DH).transpose(1, 0, 2)
    v = qkv[:, 2 * D:].reshape(S, NHEAD, DH).transpose(1, 0, 2)
    attn = _attention(q, k, v).transpose(1, 0, 2).reshape(S, D)
    x1 = _proj_res(attn, W_o, x2d)

    wr_pad = jnp.zeros((D, 128), jnp.float32).at[:, :E].set(W_r)
    h, logits = _ln2_router(x1, g2r, b2r, wr_pad)
    slot_t, gk_t = _routing(logits)

    slot3d = slot_t.T.reshape(NW, NCH, CH)
    buf = _sc_dispatch(h, slot3d)
    out_ext = _expert_ffn(buf, W1, W2)
    ycat = _sc_combine(out_ext, slot3d)

    ones = jnp.ones((1, 128), jnp.float32)
    g1 = gk_t[:, 0:1] * ones
    g2 = gk_t[:, 1:2] * ones
    y = _combine(x1, ycat[:S], ycat[S:], g1, g2)
    return y.reshape(1, S, D)


# speedup vs baseline: 1.0800x; 1.0800x over previous
import functools
import math

import jax
import jax.numpy as jnp
from jax import lax
from jax.experimental import pallas as pl
from jax.experimental.pallas import tpu as pltpu
from jax.experimental.pallas import tpu_sc as plsc

S = 2048
D = 1024
NHEAD = 16
DH = 64
E = 8
TOPK = 2
DFF = 4096
CAP = 640
TRASH = E * CAP
BUFN = TRASH + 128

ENT = TOPK * S
NW = 32
EPW = ENT // NW
CH = 64
NCH = EPW // CH

BS = 256
NEG = -1e30


def _ln(x, g, b):
    m = jnp.mean(x, axis=-1, keepdims=True)
    v = jnp.mean((x - m) ** 2, axis=-1, keepdims=True)
    return (x - m) * jax.lax.rsqrt(v + 1e-5) * g + b


def _k1_body(x_ref, g_ref, b_ref, w_ref, o_ref):
    xn = _ln(x_ref[...], g_ref[...], b_ref[...])
    o_ref[...] = jnp.dot(xn, w_ref[...], preferred_element_type=jnp.float32)


def _ln_qkv(x, g, b, w):
    return pl.pallas_call(
        _k1_body,
        grid=(S // BS, (3 * D) // 512),
        in_specs=[
            pl.BlockSpec((BS, D), lambda i, j: (i, 0)),
            pl.BlockSpec((1, D), lambda i, j: (0, 0)),
            pl.BlockSpec((1, D), lambda i, j: (0, 0)),
            pl.BlockSpec((D, 512), lambda i, j: (0, j)),
        ],
        out_specs=pl.BlockSpec((BS, 512), lambda i, j: (i, j)),
        out_shape=jax.ShapeDtypeStruct((S, 3 * D), jnp.float32),
    )(x, g, b, w)


def _k2_body(q_ref, k_ref, v_ref, o_ref):
    qi = pl.program_id(0)
    row_ids = qi * BS + lax.broadcasted_iota(jnp.int32, (BS, BS), 0)
    col_base = lax.broadcasted_iota(jnp.int32, (BS, BS), 1)
    scale = 1.0 / math.sqrt(DH)
    q1 = q_ref[:, :DH] * scale
    q2 = q_ref[:, DH:] * scale

    def step(kb, carry):
        m1, l1, a1, m2, l2, a2 = carry
        kk = k_ref[pl.ds(kb * BS, BS), :]
        vv = v_ref[pl.ds(kb * BS, BS), :]
        mask = (kb * BS + col_base) <= row_ids

        def upd(q, kh, vh, m, l, acc):
            s = lax.dot_general(q, kh, (((1,), (1,)), ((), ())),
                                preferred_element_type=jnp.float32)
            s = jnp.where(mask, s, NEG)
            m_new = jnp.maximum(m, jnp.max(s, axis=1, keepdims=True))
            p = jnp.exp(s - m_new)
            corr = jnp.exp(m - m_new)
            l_new = l * corr + jnp.sum(p, axis=1, keepdims=True)
            acc_new = acc * corr + jnp.dot(p, vh,
                                           preferred_element_type=jnp.float32)
            return m_new, l_new, acc_new

        m1, l1, a1 = upd(q1, kk[:, :DH], vv[:, :DH], m1, l1, a1)
        m2, l2, a2 = upd(q2, kk[:, DH:], vv[:, DH:], m2, l2, a2)
        return m1, l1, a1, m2, l2, a2

    z1 = jnp.full((BS, 1), NEG, jnp.float32)
    z0 = jnp.zeros((BS, 1), jnp.float32)
    za = jnp.zeros((BS, DH), jnp.float32)
    m1, l1, a1, m2, l2, a2 = lax.fori_loop(
        0, qi + 1, step, (z1, z0, za, z1, z0, za))
    o_ref[...] = jnp.concatenate([a1 / l1, a2 / l2], axis=1)


def _attention(qkv):
    nj = NHEAD // 2
    return pl.pallas_call(
        _k2_body,
        grid=(S // BS, nj),
        in_specs=[
            pl.BlockSpec((BS, 2 * DH), lambda i, j: (i, j)),
            pl.BlockSpec((S, 2 * DH), lambda i, j: (0, nj + j)),
            pl.BlockSpec((S, 2 * DH), lambda i, j: (0, 2 * nj + j)),
        ],
        out_specs=pl.BlockSpec((BS, 2 * DH), lambda i, j: (i, j)),
        out_shape=jax.ShapeDtypeStruct((S, D), jnp.float32),
    )(qkv, qkv, qkv)


def _k3_body(a_ref, w_ref, x_ref, o_ref):
    o_ref[...] = x_ref[...] + jnp.dot(a_ref[...], w_ref[...],
                                      preferred_element_type=jnp.float32)


def _proj_res(attn, w_o, x):
    return pl.pallas_call(
        _k3_body,
        grid=(S // BS, D // 256),
        in_specs=[
            pl.BlockSpec((BS, D), lambda i, j: (i, 0)),
            pl.BlockSpec((D, 256), lambda i, j: (0, j)),
            pl.BlockSpec((BS, 256), lambda i, j: (i, j)),
        ],
        out_specs=pl.BlockSpec((BS, 256), lambda i, j: (i, j)),
        out_shape=jax.ShapeDtypeStruct((S, D), jnp.float32),
    )(attn, w_o, x)


def _k4_body(x_ref, g_ref, b_ref, wr_ref, h_ref, lg_ref):
    hn = _ln(x_ref[...], g_ref[...], b_ref[...])
    h_ref[...] = hn
    lg_ref[...] = jnp.dot(hn, wr_ref[...], preferred_element_type=jnp.float32)


def _ln2_router(x1, g, b, wr_pad):
    return pl.pallas_call(
        _k4_body,
        grid=(S // BS,),
        in_specs=[
            pl.BlockSpec((BS, D), lambda i: (i, 0)),
            pl.BlockSpec((1, D), lambda i: (0, 0)),
            pl.BlockSpec((1, D), lambda i: (0, 0)),
            pl.BlockSpec((D, 128), lambda i: (0, 0)),
        ],
        out_specs=[
            pl.BlockSpec((BS, D), lambda i: (i, 0)),
            pl.BlockSpec((BS, 128), lambda i: (i, 0)),
        ],
        out_shape=[
            jax.ShapeDtypeStruct((S, D), jnp.float32),
            jax.ShapeDtypeStruct((S, 128), jnp.float32),
        ],
    )(x1, g, b, wr_pad)


def _k5_body(lg_ref, slot_ref, gk_ref):
    lane = lax.broadcasted_iota(jnp.int32, (BS, 128), 1)
    tril = (lax.broadcasted_iota(jnp.int32, (BS, BS), 0)
            > lax.broadcasted_iota(jnp.int32, (BS, BS), 1)).astype(jnp.float32)
    counts = jnp.zeros((1, 128), jnp.float32)
    for c in range(TOPK):
        for b in range(S // BS):
            raw = lg_ref[pl.ds(b * BS, BS), :]
            lg = jnp.where(lane < E, raw, NEG)
            m1 = jnp.max(lg, axis=1, keepdims=True)
            i1 = jnp.min(jnp.where(lg == m1, lane, 1000), axis=1,
                         keepdims=True)
            lg2 = jnp.where(lane == i1, NEG, lg)
            m2 = jnp.max(lg2, axis=1, keepdims=True)
            i2 = jnp.min(jnp.where(lg2 == m2, lane, 1000), axis=1,
                         keepdims=True)
            e2x = jnp.exp(m2 - m1)
            if c == 0:
                ei, gate = i1, 1.0 / (1.0 + e2x)
            else:
                ei, gate = i2, e2x / (1.0 + e2x)
            onehot = (lane == ei).astype(jnp.float32)
            run = jnp.dot(tril, onehot, preferred_element_type=jnp.float32)
            pos = jnp.sum(onehot * (counts + run), axis=1, keepdims=True)
            keep = pos < CAP
            pos_i = pos.astype(jnp.int32)
            slot = jnp.where(keep, ei * CAP + pos_i, TRASH)
            slot_ref[pl.ds(b * BS, BS), c:c + 1] = slot
            gk_ref[pl.ds(b * BS, BS), c:c + 1] = jnp.where(keep, gate, 0.0)
            counts = counts + jnp.sum(onehot, axis=0, keepdims=True)


def _routing(logits):
    return pl.pallas_call(
        _k5_body,
        in_specs=[pl.BlockSpec((S, 128), lambda: (0, 0))],
        out_specs=[
            pl.BlockSpec((S, TOPK), lambda: (0, 0)),
            pl.BlockSpec((S, TOPK), lambda: (0, 0)),
        ],
        out_shape=[
            jax.ShapeDtypeStruct((S, TOPK), jnp.int32),
            jax.ShapeDtypeStruct((S, TOPK), jnp.float32),
        ],
    )(logits)


def _sc_dispatch_body(h_hbm, slot_hbm, buf_hbm, idx_v, rows_v, sem):
    wid = lax.axis_index("s") * 2 + lax.axis_index("c")
    for ch in range(NCH):
        base = wid * EPW + ch * CH
        tok = base % S
        pltpu.sync_copy(slot_hbm.at[wid, ch], idx_v)
        pltpu.sync_copy(h_hbm.at[pl.ds(tok, CH)], rows_v)
        pltpu.async_copy(rows_v, buf_hbm.at[idx_v], sem).wait()


def _sc_dispatch(h, slot3d):
    kfn = pl.kernel(
        _sc_dispatch_body,
        out_type=jax.ShapeDtypeStruct((BUFN, D), jnp.float32),
        scratch_types=[
            pltpu.VMEM((CH,), jnp.int32),
            pltpu.VMEM((CH, D), jnp.float32),
            pltpu.SemaphoreType.DMA,
        ],
        mesh=plsc.VectorSubcoreMesh(core_axis_name="c", subcore_axis_name="s"),
    )
    return kfn(h, slot3d)


def _k7_body(buf_ref, w1_ref, w2_ref, o_ref, h1_ref):
    f = pl.program_id(1)
    h1 = jnp.dot(buf_ref[...].astype(jnp.bfloat16), w1_ref[0],
                 preferred_element_type=jnp.float32)
    h1_ref[...] = jax.nn.gelu(h1, approximate=True).astype(jnp.bfloat16)

    @pl.when(f == 0)
    def _():
        o_ref[...] = jnp.zeros_like(o_ref)

    o_ref[...] += jnp.dot(h1_ref[...], w2_ref[0],
                          preferred_element_type=jnp.float32)


def _expert_ffn(buf, w1, w2):
    fsp = DFF // 2
    return pl.pallas_call(
        _k7_body,
        grid=(E, 2),
        in_specs=[
            pl.BlockSpec((CAP, D), lambda e, f: (e, 0)),
            pl.BlockSpec((1, D, fsp), lambda e, f: (e, 0, f)),
            pl.BlockSpec((1, fsp, D), lambda e, f: (e, f, 0)),
        ],
        out_specs=pl.BlockSpec((CAP, D), lambda e, f: (e, 0)),
        out_shape=jax.ShapeDtypeStruct((BUFN, D), jnp.float32),
        scratch_shapes=[pltpu.VMEM((CAP, fsp), jnp.bfloat16)],
    )(buf, w1, w2)


def _sc_combine_body(oute_hbm, slot_hbm, ycat_hbm, idx_v, rows_v, sem):
    wid = lax.axis_index("s") * 2 + lax.axis_index("c")
    for ch in range(NCH):
        base = wid * EPW + ch * CH
        pltpu.sync_copy(slot_hbm.at[wid, ch], idx_v)
        pltpu.async_copy(oute_hbm.at[idx_v], rows_v, sem).wait()
        pltpu.sync_copy(rows_v, ycat_hbm.at[pl.ds(base, CH)])


def _sc_combine(out_ext, slot3d):
    kfn = pl.kernel(
        _sc_combine_body,
        out_type=jax.ShapeDtypeStruct((ENT, D), jnp.float32),
        scratch_types=[
            pltpu.VMEM((CH,), jnp.int32),
            pltpu.VMEM((CH, D), jnp.float32),
            pltpu.SemaphoreType.DMA,
        ],
        mesh=plsc.VectorSubcoreMesh(core_axis_name="c", subcore_axis_name="s"),
    )
    return kfn(out_ext, slot3d)


def _k9_body(x_ref, y1_ref, y2_ref, g1_ref, g2_ref, o_ref):
    g1 = g1_ref[...][:, 0:1]
    g2 = g2_ref[...][:, 0:1]
    y1 = jnp.where(g1 > 0, y1_ref[...] * g1, 0.0)
    y2 = jnp.where(g2 > 0, y2_ref[...] * g2, 0.0)
    o_ref[...] = x_ref[...] + y1 + y2


def _combine(x1, y1, y2, g1, g2):
    return pl.pallas_call(
        _k9_body,
        grid=(S // BS,),
        in_specs=[
            pl.BlockSpec((BS, D), lambda i: (i, 0)),
            pl.BlockSpec((BS, D), lambda i: (i, 0)),
            pl.BlockSpec((BS, D), lambda i: (i, 0)),
            pl.BlockSpec((BS, 128), lambda i: (i, 0)),
            pl.BlockSpec((BS, 128), lambda i: (i, 0)),
        ],
        out_specs=pl.BlockSpec((BS, D), lambda i: (i, 0)),
        out_shape=jax.ShapeDtypeStruct((S, D), jnp.float32),
    )(x1, y1, y2, g1, g2)


def kernel(x, ln1_g, ln1_b, W_qkv, W_o, ln2_g, ln2_b, W_r, W1, W2):
    x2d = x.reshape(S, D)
    g1r = ln1_g.reshape(1, D)
    b1r = ln1_b.reshape(1, D)
    g2r = ln2_g.reshape(1, D)
    b2r = ln2_b.reshape(1, D)

    qkv = _ln_qkv(x2d, g1r, b1r, W_qkv)
    attn = _attention(qkv)
    x1 = _proj_res(attn, W_o, x2d)

    wr_pad = jnp.zeros((D, 128), jnp.float32).at[:, :E].set(W_r)
    h, logits = _ln2_router(x1, g2r, b2r, wr_pad)
    slot_t, gk_t = _routing(logits)

    slot3d = slot_t.T.reshape(NW, NCH, CH)
    buf = _sc_dispatch(h, slot3d)
    out_ext = _expert_ffn(buf, W1.astype(jnp.bfloat16), W2.astype(jnp.bfloat16))
    ycat = _sc_combine(out_ext, slot3d)

    ones = jnp.ones((1, 128), jnp.float32)
    g1 = gk_t[:, 0:1] * ones
    g2 = gk_t[:, 1:2] * ones
    y = _combine(x1, ycat[:S], ycat[S:], g1, g2)
    return y.reshape(1, S, D)


# speedup vs baseline: 1.4471x; 1.3399x over previous
import functools
import math

import jax
import jax.numpy as jnp
from jax import lax
from jax.experimental import pallas as pl
from jax.experimental.pallas import tpu as pltpu
from jax.experimental.pallas import tpu_sc as plsc

S = 2048
D = 1024
NHEAD = 16
DH = 64
E = 8
TOPK = 2
DFF = 4096
CAP = 640
TRASH = E * CAP
BUFN = TRASH + 128

ENT = TOPK * S
NW = 32
EPW = ENT // NW
CH = 64
NCH = EPW // CH

BS = 256
NEG = -1e30


def _ln(x, g, b):
    m = jnp.mean(x, axis=-1, keepdims=True)
    v = jnp.mean((x - m) ** 2, axis=-1, keepdims=True)
    return (x - m) * jax.lax.rsqrt(v + 1e-5) * g + b


def _k1_body(x_ref, g_ref, b_ref, w_ref, o_ref):
    xn = _ln(x_ref[...], g_ref[...], b_ref[...])
    o_ref[...] = jnp.dot(xn, w_ref[...], preferred_element_type=jnp.float32)


def _ln_qkv(x, g, b, w):
    return pl.pallas_call(
        _k1_body,
        grid=(S // BS,),
        in_specs=[
            pl.BlockSpec((BS, D), lambda i: (i, 0)),
            pl.BlockSpec((1, D), lambda i: (0, 0)),
            pl.BlockSpec((1, D), lambda i: (0, 0)),
            pl.BlockSpec((D, 3 * D), lambda i: (0, 0)),
        ],
        out_specs=pl.BlockSpec((BS, 3 * D), lambda i: (i, 0)),
        out_shape=jax.ShapeDtypeStruct((S, 3 * D), jnp.float32),
    )(x, g, b, w)


def _k2_body(q_ref, k_ref, v_ref, o_ref, s1_scr, s2_scr):
    qi = pl.program_id(1)
    scale = 1.0 / math.sqrt(DH)
    q1 = q_ref[:, :DH] * scale
    q2 = q_ref[:, DH:] * scale

    def score(kb, carry, mask):
        m1, m2 = carry
        kk = k_ref[pl.ds(kb * BS, BS), :]
        s1 = lax.dot_general(q1, kk[:, :DH], (((1,), (1,)), ((), ())),
                             preferred_element_type=jnp.float32)
        s2 = lax.dot_general(q2, kk[:, DH:], (((1,), (1,)), ((), ())),
                             preferred_element_type=jnp.float32)
        if mask is not None:
            s1 = jnp.where(mask, s1, NEG)
            s2 = jnp.where(mask, s2, NEG)
        s1_scr[:, pl.ds(kb * BS, BS)] = s1
        s2_scr[:, pl.ds(kb * BS, BS)] = s2
        m1 = jnp.maximum(m1, jnp.max(s1, axis=1, keepdims=True))
        m2 = jnp.maximum(m2, jnp.max(s2, axis=1, keepdims=True))
        return m1, m2

    z1 = jnp.full((BS, 1), NEG, jnp.float32)
    m1, m2 = lax.fori_loop(0, qi, lambda kb, c: score(kb, c, None), (z1, z1))
    dmask = (lax.broadcasted_iota(jnp.int32, (BS, BS), 0)
             >= lax.broadcasted_iota(jnp.int32, (BS, BS), 1))
    m1, m2 = score(qi, (m1, m2), dmask)

    def accum(kb, carry):
        l1, a1, l2, a2 = carry
        vv = v_ref[pl.ds(kb * BS, BS), :]
        p1 = jnp.exp(s1_scr[:, pl.ds(kb * BS, BS)] - m1)
        p2 = jnp.exp(s2_scr[:, pl.ds(kb * BS, BS)] - m2)
        l1 = l1 + jnp.sum(p1, axis=1, keepdims=True)
        l2 = l2 + jnp.sum(p2, axis=1, keepdims=True)
        a1 = a1 + jnp.dot(p1, vv[:, :DH], preferred_element_type=jnp.float32)
        a2 = a2 + jnp.dot(p2, vv[:, DH:], preferred_element_type=jnp.float32)
        return l1, a1, l2, a2

    z0 = jnp.zeros((BS, 1), jnp.float32)
    za = jnp.zeros((BS, DH), jnp.float32)
    l1, a1, l2, a2 = lax.fori_loop(0, qi + 1, accum, (z0, za, z0, za))
    o_ref[...] = jnp.concatenate([a1 / l1, a2 / l2], axis=1)


def _attention(qkv):
    nj = NHEAD // 2
    return pl.pallas_call(
        _k2_body,
        grid=(nj, S // BS),
        in_specs=[
            pl.BlockSpec((BS, 2 * DH), lambda j, i: (i, j)),
            pl.BlockSpec((S, 2 * DH), lambda j, i: (0, nj + j)),
            pl.BlockSpec((S, 2 * DH), lambda j, i: (0, 2 * nj + j)),
        ],
        out_specs=pl.BlockSpec((BS, 2 * DH), lambda j, i: (i, j)),
        out_shape=jax.ShapeDtypeStruct((S, D), jnp.float32),
        scratch_shapes=[
            pltpu.VMEM((BS, S), jnp.float32),
            pltpu.VMEM((BS, S), jnp.float32),
        ],
    )(qkv, qkv, qkv)


def _k3_body(a_ref, wo_ref, x_ref, g_ref, b_ref, wr_ref,
             x1_ref, h_ref, lg_ref):
    x1 = x_ref[...] + jnp.dot(a_ref[...], wo_ref[...],
                              preferred_element_type=jnp.float32)
    x1_ref[...] = x1
    hn = _ln(x1, g_ref[...], b_ref[...])
    h_ref[...] = hn
    lg_ref[...] = jnp.dot(hn, wr_ref[...], preferred_element_type=jnp.float32)


def _proj_ln2_router(attn, w_o, x, g, b, wr_pad):
    return pl.pallas_call(
        _k3_body,
        grid=(S // BS,),
        in_specs=[
            pl.BlockSpec((BS, D), lambda i: (i, 0)),
            pl.BlockSpec((D, D), lambda i: (0, 0)),
            pl.BlockSpec((BS, D), lambda i: (i, 0)),
            pl.BlockSpec((1, D), lambda i: (0, 0)),
            pl.BlockSpec((1, D), lambda i: (0, 0)),
            pl.BlockSpec((D, 128), lambda i: (0, 0)),
        ],
        out_specs=[
            pl.BlockSpec((BS, D), lambda i: (i, 0)),
            pl.BlockSpec((BS, D), lambda i: (i, 0)),
            pl.BlockSpec((BS, 128), lambda i: (i, 0)),
        ],
        out_shape=[
            jax.ShapeDtypeStruct((S, D), jnp.float32),
            jax.ShapeDtypeStruct((S, D), jnp.float32),
            jax.ShapeDtypeStruct((S, 128), jnp.float32),
        ],
    )(attn, w_o, x, g, b, wr_pad)


def _k5_body(lg_ref, slot_ref, gk_ref):
    lane = lax.broadcasted_iota(jnp.int32, (BS, 128), 1)
    tril = (lax.broadcasted_iota(jnp.int32, (BS, BS), 0)
            > lax.broadcasted_iota(jnp.int32, (BS, BS), 1)).astype(jnp.float32)
    counts = jnp.zeros((1, 128), jnp.float32)
    for c in range(TOPK):
        for b in range(S // BS):
            raw = lg_ref[pl.ds(b * BS, BS), :]
            lg = jnp.where(lane < E, raw, NEG)
            m1 = jnp.max(lg, axis=1, keepdims=True)
            i1 = jnp.min(jnp.where(lg == m1, lane, 1000), axis=1,
                         keepdims=True)
            lg2 = jnp.where(lane == i1, NEG, lg)
            m2 = jnp.max(lg2, axis=1, keepdims=True)
            i2 = jnp.min(jnp.where(lg2 == m2, lane, 1000), axis=1,
                         keepdims=True)
            e2x = jnp.exp(m2 - m1)
            if c == 0:
                ei, gate = i1, 1.0 / (1.0 + e2x)
            else:
                ei, gate = i2, e2x / (1.0 + e2x)
            onehot = (lane == ei).astype(jnp.float32)
            run = jnp.dot(tril, onehot, preferred_element_type=jnp.float32)
            pos = jnp.sum(onehot * (counts + run), axis=1, keepdims=True)
            keep = pos < CAP
            pos_i = pos.astype(jnp.int32)
            slot = jnp.where(keep, ei * CAP + pos_i, TRASH)
            slot_ref[pl.ds(b * BS, BS), c:c + 1] = slot
            gk_ref[pl.ds(b * BS, BS), c:c + 1] = jnp.where(keep, gate, 0.0)
            counts = counts + jnp.sum(onehot, axis=0, keepdims=True)


def _routing(logits):
    return pl.pallas_call(
        _k5_body,
        in_specs=[pl.BlockSpec((S, 128), lambda: (0, 0))],
        out_specs=[
            pl.BlockSpec((S, TOPK), lambda: (0, 0)),
            pl.BlockSpec((S, TOPK), lambda: (0, 0)),
        ],
        out_shape=[
            jax.ShapeDtypeStruct((S, TOPK), jnp.int32),
            jax.ShapeDtypeStruct((S, TOPK), jnp.float32),
        ],
    )(logits)


def _sc_dispatch_body(h_hbm, slot_hbm, buf_hbm, idx_v, rows_v, sem):
    wid = lax.axis_index("s") * 2 + lax.axis_index("c")
    for ch in range(NCH):
        base = wid * EPW + ch * CH
        tok = base % S
        pltpu.sync_copy(slot_hbm.at[wid, ch], idx_v)
        pltpu.sync_copy(h_hbm.at[pl.ds(tok, CH)], rows_v)
        pltpu.async_copy(rows_v, buf_hbm.at[idx_v], sem).wait()


def _sc_dispatch(h, slot3d):
    kfn = pl.kernel(
        _sc_dispatch_body,
        out_type=jax.ShapeDtypeStruct((BUFN, D), jnp.float32),
        scratch_types=[
            pltpu.VMEM((CH,), jnp.int32),
            pltpu.VMEM((CH, D), jnp.float32),
            pltpu.SemaphoreType.DMA,
        ],
        mesh=plsc.VectorSubcoreMesh(core_axis_name="c", subcore_axis_name="s"),
    )
    return kfn(h, slot3d)


FSP = DFF // 2
FCH = 512


def _k7_body(buf_ref, w1_ref, w2_ref, o_ref):
    f = pl.program_id(1)
    acc = None
    for c in range(FSP // FCH):
        w1c = w1_ref[0, :, c * FCH:(c + 1) * FCH].astype(jnp.bfloat16)
        h1 = jnp.dot(buf_ref[...].astype(jnp.bfloat16), w1c,
                     preferred_element_type=jnp.float32)
        g = jax.nn.gelu(h1, approximate=True).astype(jnp.bfloat16)
        w2c = w2_ref[0, c * FCH:(c + 1) * FCH, :].astype(jnp.bfloat16)
        part = jnp.dot(g, w2c, preferred_element_type=jnp.float32)
        acc = part if acc is None else acc + part

    @pl.when(f == 0)
    def _():
        o_ref[...] = acc

    @pl.when(f != 0)
    def _():
        o_ref[...] += acc


def _expert_ffn(buf, w1, w2):
    return pl.pallas_call(
        _k7_body,
        grid=(E, 2),
        in_specs=[
            pl.BlockSpec((CAP, D), lambda e, f: (e, 0)),
            pl.BlockSpec((1, D, FSP), lambda e, f: (e, 0, f)),
            pl.BlockSpec((1, FSP, D), lambda e, f: (e, f, 0)),
        ],
        out_specs=pl.BlockSpec((CAP, D), lambda e, f: (e, 0)),
        out_shape=jax.ShapeDtypeStruct((BUFN, D), jnp.float32),
    )(buf, w1, w2)


def _sc_combine_body(oute_hbm, slot_hbm, ycat_hbm, idx_v, rows_v, sem):
    wid = lax.axis_index("s") * 2 + lax.axis_index("c")
    for ch in range(NCH):
        base = wid * EPW + ch * CH
        pltpu.sync_copy(slot_hbm.at[wid, ch], idx_v)
        pltpu.async_copy(oute_hbm.at[idx_v], rows_v, sem).wait()
        pltpu.sync_copy(rows_v, ycat_hbm.at[pl.ds(base, CH)])


def _sc_combine(out_ext, slot3d):
    kfn = pl.kernel(
        _sc_combine_body,
        out_type=jax.ShapeDtypeStruct((ENT, D), jnp.float32),
        scratch_types=[
            pltpu.VMEM((CH,), jnp.int32),
            pltpu.VMEM((CH, D), jnp.float32),
            pltpu.SemaphoreType.DMA,
        ],
        mesh=plsc.VectorSubcoreMesh(core_axis_name="c", subcore_axis_name="s"),
    )
    return kfn(out_ext, slot3d)


def _k9_body(x_ref, y1_ref, y2_ref, g1_ref, g2_ref, o_ref):
    g1 = g1_ref[...][:, 0:1]
    g2 = g2_ref[...][:, 0:1]
    y1 = jnp.where(g1 > 0, y1_ref[...] * g1, 0.0)
    y2 = jnp.where(g2 > 0, y2_ref[...] * g2, 0.0)
    o_ref[...] = x_ref[...] + y1 + y2


def _combine(x1, y1, y2, g1, g2):
    return pl.pallas_call(
        _k9_body,
        grid=(S // BS,),
        in_specs=[
            pl.BlockSpec((BS, D), lambda i: (i, 0)),
            pl.BlockSpec((BS, D), lambda i: (i, 0)),
            pl.BlockSpec((BS, D), lambda i: (i, 0)),
            pl.BlockSpec((BS, 128), lambda i: (i, 0)),
            pl.BlockSpec((BS, 128), lambda i: (i, 0)),
        ],
        out_specs=pl.BlockSpec((BS, D), lambda i: (i, 0)),
        out_shape=jax.ShapeDtypeStruct((S, D), jnp.float32),
    )(x1, y1, y2, g1, g2)


def kernel(x, ln1_g, ln1_b, W_qkv, W_o, ln2_g, ln2_b, W_r, W1, W2):
    x2d = x.reshape(S, D)
    g1r = ln1_g.reshape(1, D)
    b1r = ln1_b.reshape(1, D)
    g2r = ln2_g.reshape(1, D)
    b2r = ln2_b.reshape(1, D)

    qkv = _ln_qkv(x2d, g1r, b1r, W_qkv)
    attn = _attention(qkv)

    wr_pad = jnp.zeros((D, 128), jnp.float32).at[:, :E].set(W_r)
    x1, h, logits = _proj_ln2_router(attn, W_o, x2d, g2r, b2r, wr_pad)
    slot_t, gk_t = _routing(logits)

    slot3d = slot_t.T.reshape(NW, NCH, CH)
    buf = _sc_dispatch(h, slot3d)
    out_ext = _expert_ffn(buf, W1, W2)
    ycat = _sc_combine(out_ext, slot3d)

    ones = jnp.ones((1, 128), jnp.float32)
    g1 = gk_t[:, 0:1] * ones
    g2 = gk_t[:, 1:2] * ones
    y = _combine(x1, ycat[:S], ycat[S:], g1, g2)
    return y.reshape(1, S, D)


# speedup vs baseline: 1.6804x; 1.1612x over previous
import functools
import math

import jax
import jax.numpy as jnp
from jax import lax
from jax.experimental import pallas as pl
from jax.experimental.pallas import tpu as pltpu
from jax.experimental.pallas import tpu_sc as plsc

S = 2048
D = 1024
NHEAD = 16
DH = 64
E = 8
TOPK = 2
DFF = 4096
CAP = 640
TRASH = E * CAP
BUFN = TRASH + 128

ENT = TOPK * S
NW = 32
EPW = ENT // NW
CH = 64
NCH = EPW // CH

BS = 256
NEG = -1e30


def _ln(x, g, b):
    m = jnp.mean(x, axis=-1, keepdims=True)
    v = jnp.mean((x - m) ** 2, axis=-1, keepdims=True)
    return (x - m) * jax.lax.rsqrt(v + 1e-5) * g + b


def _k1_body(x_ref, g_ref, b_ref, w_ref, o_ref):
    xn = _ln(x_ref[...], g_ref[...], b_ref[...])
    o_ref[...] = jnp.dot(xn, w_ref[...], preferred_element_type=jnp.float32)


def _ln_qkv(x, g, b, w):
    return pl.pallas_call(
        _k1_body,
        grid=(S // BS,),
        in_specs=[
            pl.BlockSpec((BS, D), lambda i: (i, 0)),
            pl.BlockSpec((1, D), lambda i: (0, 0)),
            pl.BlockSpec((1, D), lambda i: (0, 0)),
            pl.BlockSpec((D, 3 * D), lambda i: (0, 0)),
        ],
        out_specs=pl.BlockSpec((BS, 3 * D), lambda i: (i, 0)),
        out_shape=jax.ShapeDtypeStruct((S, 3 * D), jnp.float32),
    )(x, g, b, w)


QBS = 512
KBS = 256


def _k2_body(q_ref, k_ref, v_ref, o_ref, s1_scr, s2_scr):
    qi = pl.program_id(1)
    scale = 1.0 / math.sqrt(DH)
    q1 = q_ref[:, :DH] * scale
    q2 = q_ref[:, DH:] * scale

    def score(kb, carry, mask):
        m1, m2 = carry
        kk = k_ref[pl.ds(kb * KBS, KBS), :]
        s1 = lax.dot_general(q1, kk[:, :DH], (((1,), (1,)), ((), ())),
                             preferred_element_type=jnp.float32)
        s2 = lax.dot_general(q2, kk[:, DH:], (((1,), (1,)), ((), ())),
                             preferred_element_type=jnp.float32)
        if mask is not None:
            s1 = jnp.where(mask, s1, NEG)
            s2 = jnp.where(mask, s2, NEG)
        s1_scr[:, pl.ds(kb * KBS, KBS)] = s1
        s2_scr[:, pl.ds(kb * KBS, KBS)] = s2
        m1 = jnp.maximum(m1, jnp.max(s1, axis=1, keepdims=True))
        m2 = jnp.maximum(m2, jnp.max(s2, axis=1, keepdims=True))
        return m1, m2

    nfull = 2 * qi
    z1 = jnp.full((QBS, 1), NEG, jnp.float32)
    carry = lax.fori_loop(0, nfull, lambda kb, c: score(kb, c, None),
                          (z1, z1))
    rl = lax.broadcasted_iota(jnp.int32, (QBS, KBS), 0)
    cl = lax.broadcasted_iota(jnp.int32, (QBS, KBS), 1)
    carry = score(nfull, carry, rl >= cl)
    m1, m2 = score(nfull + 1, carry, rl >= cl + KBS)

    def accum(kb, carry):
        l1, a1, l2, a2 = carry
        vv = v_ref[pl.ds(kb * KBS, KBS), :]
        p1 = jnp.exp(s1_scr[:, pl.ds(kb * KBS, KBS)] - m1)
        p2 = jnp.exp(s2_scr[:, pl.ds(kb * KBS, KBS)] - m2)
        l1 = l1 + jnp.sum(p1, axis=1, keepdims=True)
        l2 = l2 + jnp.sum(p2, axis=1, keepdims=True)
        a1 = a1 + jnp.dot(p1, vv[:, :DH], preferred_element_type=jnp.float32)
        a2 = a2 + jnp.dot(p2, vv[:, DH:], preferred_element_type=jnp.float32)
        return l1, a1, l2, a2

    z0 = jnp.zeros((QBS, 1), jnp.float32)
    za = jnp.zeros((QBS, DH), jnp.float32)
    l1, a1, l2, a2 = lax.fori_loop(0, nfull + 2, accum, (z0, za, z0, za))
    o_ref[...] = jnp.concatenate([a1 / l1, a2 / l2], axis=1)


def _attention(qkv):
    nj = NHEAD // 2
    return pl.pallas_call(
        _k2_body,
        grid=(nj, S // QBS),
        in_specs=[
            pl.BlockSpec((QBS, 2 * DH), lambda j, i: (i, j)),
            pl.BlockSpec((S, 2 * DH), lambda j, i: (0, nj + j)),
            pl.BlockSpec((S, 2 * DH), lambda j, i: (0, 2 * nj + j)),
        ],
        out_specs=pl.BlockSpec((QBS, 2 * DH), lambda j, i: (i, j)),
        out_shape=jax.ShapeDtypeStruct((S, D), jnp.float32),
        scratch_shapes=[
            pltpu.VMEM((QBS, S), jnp.float32),
            pltpu.VMEM((QBS, S), jnp.float32),
        ],
    )(qkv, qkv, qkv)


def _k3_body(a_ref, wo_ref, x_ref, g_ref, b_ref, wr_ref,
             x1_ref, h_ref, lg_ref):
    x1 = x_ref[...] + jnp.dot(a_ref[...], wo_ref[...],
                              preferred_element_type=jnp.float32)
    x1_ref[...] = x1
    hn = _ln(x1, g_ref[...], b_ref[...])
    h_ref[...] = hn
    lg_ref[...] = jnp.dot(hn, wr_ref[...], preferred_element_type=jnp.float32)


def _proj_ln2_router(attn, w_o, x, g, b, wr_pad):
    return pl.pallas_call(
        _k3_body,
        grid=(S // BS,),
        in_specs=[
            pl.BlockSpec((BS, D), lambda i: (i, 0)),
            pl.BlockSpec((D, D), lambda i: (0, 0)),
            pl.BlockSpec((BS, D), lambda i: (i, 0)),
            pl.BlockSpec((1, D), lambda i: (0, 0)),
            pl.BlockSpec((1, D), lambda i: (0, 0)),
            pl.BlockSpec((D, 128), lambda i: (0, 0)),
        ],
        out_specs=[
            pl.BlockSpec((BS, D), lambda i: (i, 0)),
            pl.BlockSpec((BS, D), lambda i: (i, 0)),
            pl.BlockSpec((BS, 128), lambda i: (i, 0)),
        ],
        out_shape=[
            jax.ShapeDtypeStruct((S, D), jnp.float32),
            jax.ShapeDtypeStruct((S, D), jnp.float32),
            jax.ShapeDtypeStruct((S, 128), jnp.float32),
        ],
    )(attn, w_o, x, g, b, wr_pad)


def _k5_body(lg_ref, slot_ref, gk_ref):
    lane = lax.broadcasted_iota(jnp.int32, (BS, 128), 1)
    tril = (lax.broadcasted_iota(jnp.int32, (BS, BS), 0)
            > lax.broadcasted_iota(jnp.int32, (BS, BS), 1)).astype(jnp.float32)
    counts = jnp.zeros((1, 128), jnp.float32)
    for c in range(TOPK):
        for b in range(S // BS):
            raw = lg_ref[pl.ds(b * BS, BS), :]
            lg = jnp.where(lane < E, raw, NEG)
            m1 = jnp.max(lg, axis=1, keepdims=True)
            i1 = jnp.min(jnp.where(lg == m1, lane, 1000), axis=1,
                         keepdims=True)
            lg2 = jnp.where(lane == i1, NEG, lg)
            m2 = jnp.max(lg2, axis=1, keepdims=True)
            i2 = jnp.min(jnp.where(lg2 == m2, lane, 1000), axis=1,
                         keepdims=True)
            e2x = jnp.exp(m2 - m1)
            if c == 0:
                ei, gate = i1, 1.0 / (1.0 + e2x)
            else:
                ei, gate = i2, e2x / (1.0 + e2x)
            onehot = (lane == ei).astype(jnp.float32)
            run = jnp.dot(tril, onehot, preferred_element_type=jnp.float32)
            pos = jnp.sum(onehot * (counts + run), axis=1, keepdims=True)
            keep = pos < CAP
            pos_i = pos.astype(jnp.int32)
            slot = jnp.where(keep, ei * CAP + pos_i, TRASH)
            slot_ref[pl.ds(b * BS, BS), c:c + 1] = slot
            gk_ref[pl.ds(b * BS, BS), c:c + 1] = jnp.where(keep, gate, 0.0)
            counts = counts + jnp.sum(onehot, axis=0, keepdims=True)


def _routing(logits):
    return pl.pallas_call(
        _k5_body,
        in_specs=[pl.BlockSpec((S, 128), lambda: (0, 0))],
        out_specs=[
            pl.BlockSpec((S, TOPK), lambda: (0, 0)),
            pl.BlockSpec((S, TOPK), lambda: (0, 0)),
        ],
        out_shape=[
            jax.ShapeDtypeStruct((S, TOPK), jnp.int32),
            jax.ShapeDtypeStruct((S, TOPK), jnp.float32),
        ],
    )(logits)


def _sc_dispatch_body(h_hbm, slot_hbm, buf_hbm, idx_v, rows_v, sem):
    wid = lax.axis_index("s") * 2 + lax.axis_index("c")
    for ch in range(NCH):
        base = wid * EPW + ch * CH
        tok = base % S
        pltpu.sync_copy(slot_hbm.at[wid, ch], idx_v)
        pltpu.sync_copy(h_hbm.at[pl.ds(tok, CH)], rows_v)
        pltpu.async_copy(rows_v, buf_hbm.at[idx_v], sem).wait()


def _sc_dispatch(h, slot3d):
    kfn = pl.kernel(
        _sc_dispatch_body,
        out_type=jax.ShapeDtypeStruct((BUFN, D), jnp.float32),
        scratch_types=[
            pltpu.VMEM((CH,), jnp.int32),
            pltpu.VMEM((CH, D), jnp.float32),
            pltpu.SemaphoreType.DMA,
        ],
        mesh=plsc.VectorSubcoreMesh(core_axis_name="c", subcore_axis_name="s"),
    )
    return kfn(h, slot3d)


FSP = DFF // 2
FCH = 512


def _k7_body(buf_ref, w1_ref, w2_ref, o_ref):
    f = pl.program_id(1)
    acc = None
    for c in range(FSP // FCH):
        w1c = w1_ref[0, :, c * FCH:(c + 1) * FCH].astype(jnp.bfloat16)
        h1 = jnp.dot(buf_ref[...].astype(jnp.bfloat16), w1c,
                     preferred_element_type=jnp.float32)
        g = jax.nn.gelu(h1, approximate=True).astype(jnp.bfloat16)
        w2c = w2_ref[0, c * FCH:(c + 1) * FCH, :].astype(jnp.bfloat16)
        part = jnp.dot(g, w2c, preferred_element_type=jnp.float32)
        acc = part if acc is None else acc + part

    @pl.when(f == 0)
    def _():
        o_ref[...] = acc

    @pl.when(f != 0)
    def _():
        o_ref[...] += acc


def _expert_ffn(buf, w1, w2):
    return pl.pallas_call(
        _k7_body,
        grid=(E, 2),
        in_specs=[
            pl.BlockSpec((CAP, D), lambda e, f: (e, 0)),
            pl.BlockSpec((1, D, FSP), lambda e, f: (e, 0, f)),
            pl.BlockSpec((1, FSP, D), lambda e, f: (e, f, 0)),
        ],
        out_specs=pl.BlockSpec((CAP, D), lambda e, f: (e, 0)),
        out_shape=jax.ShapeDtypeStruct((BUFN, D), jnp.float32),
    )(buf, w1, w2)


def _sc_combine_body(oute_hbm, slot_hbm, ycat_hbm, idx_v, rows_v, sem):
    wid = lax.axis_index("s") * 2 + lax.axis_index("c")
    for ch in range(NCH):
        base = wid * EPW + ch * CH
        pltpu.sync_copy(slot_hbm.at[wid, ch], idx_v)
        pltpu.async_copy(oute_hbm.at[idx_v], rows_v, sem).wait()
        pltpu.sync_copy(rows_v, ycat_hbm.at[pl.ds(base, CH)])


def _sc_combine(out_ext, slot3d):
    kfn = pl.kernel(
        _sc_combine_body,
        out_type=jax.ShapeDtypeStruct((ENT, D), jnp.float32),
        scratch_types=[
            pltpu.VMEM((CH,), jnp.int32),
            pltpu.VMEM((CH, D), jnp.float32),
            pltpu.SemaphoreType.DMA,
        ],
        mesh=plsc.VectorSubcoreMesh(core_axis_name="c", subcore_axis_name="s"),
    )
    return kfn(out_ext, slot3d)


def _k9_body(x_ref, y1_ref, y2_ref, g1_ref, g2_ref, o_ref):
    g1 = g1_ref[...][:, 0:1]
    g2 = g2_ref[...][:, 0:1]
    y1 = jnp.where(g1 > 0, y1_ref[...] * g1, 0.0)
    y2 = jnp.where(g2 > 0, y2_ref[...] * g2, 0.0)
    o_ref[...] = x_ref[...] + y1 + y2


def _combine(x1, y1, y2, g1, g2):
    return pl.pallas_call(
        _k9_body,
        grid=(S // BS,),
        in_specs=[
            pl.BlockSpec((BS, D), lambda i: (i, 0)),
            pl.BlockSpec((BS, D), lambda i: (i, 0)),
            pl.BlockSpec((BS, D), lambda i: (i, 0)),
            pl.BlockSpec((BS, 128), lambda i: (i, 0)),
            pl.BlockSpec((BS, 128), lambda i: (i, 0)),
        ],
        out_specs=pl.BlockSpec((BS, D), lambda i: (i, 0)),
        out_shape=jax.ShapeDtypeStruct((S, D), jnp.float32),
    )(x1, y1, y2, g1, g2)


def kernel(x, ln1_g, ln1_b, W_qkv, W_o, ln2_g, ln2_b, W_r, W1, W2):
    x2d = x.reshape(S, D)
    g1r = ln1_g.reshape(1, D)
    b1r = ln1_b.reshape(1, D)
    g2r = ln2_g.reshape(1, D)
    b2r = ln2_b.reshape(1, D)

    qkv = _ln_qkv(x2d, g1r, b1r, W_qkv)
    attn = _attention(qkv)

    wr_pad = jnp.zeros((D, 128), jnp.float32).at[:, :E].set(W_r)
    x1, h, logits = _proj_ln2_router(attn, W_o, x2d, g2r, b2r, wr_pad)
    slot_t, gk_t = _routing(logits)

    slot3d = slot_t.T.reshape(NW, NCH, CH)
    buf = _sc_dispatch(h, slot3d)
    out_ext = _expert_ffn(buf, W1, W2)
    ycat = _sc_combine(out_ext, slot3d)

    ones = jnp.ones((1, 128), jnp.float32)
    g1 = gk_t[:, 0:1] * ones
    g2 = gk_t[:, 1:2] * ones
    y = _combine(x1, ycat[:S], ycat[S:], g1, g2)
    return y.reshape(1, S, D)
